# baseline (speedup 1.0000x reference)
import numpy as np
import jax
import jax.numpy as jnp
from jax import lax
from jax.experimental import pallas as pl
from jax.experimental.pallas import tpu as pltpu
from jax.experimental.pallas import tpu_sc as plsc

VOCAB = 50272
LANES = 16
NTILES = 16
UNR = 4
MAIN_VECS = 192
MAIN = MAIN_VECS * LANES
MAIN_TOTAL = NTILES * MAIN
REM_BASE = MAIN_TOTAL
REM_TOTAL = VOCAB - MAIN_TOTAL
REM_VECS = REM_TOTAL // LANES
REM_PER_TILE = 5
REM_LAST = (VOCAB // 128) * 128
SROW = 4 * LANES

_ROTS = ((13, 15, 26, 6), (17, 29, 16, 24))
_SCHED = ((1, 2, 1), (2, 0, 2), (0, 1, 3), (1, 2, 4), (2, 0, 5))


def _np_threefry_bits(seed_u32: int, counts_lo: np.ndarray) -> np.ndarray:
    x0 = np.zeros_like(counts_lo, dtype=np.uint32)
    x1 = counts_lo.astype(np.uint32)
    ks = [np.uint32(0), np.uint32(seed_u32),
          np.uint32(np.uint32(seed_u32) ^ np.uint32(0x1BD11BDA))]
    x0 = (x0 + ks[0]).astype(np.uint32)
    x1 = (x1 + ks[1]).astype(np.uint32)
    for g in range(5):
        for r in _ROTS[g % 2]:
            x0 = (x0 + x1).astype(np.uint32)
            x1 = ((x1 << np.uint32(r)) | (x1 >> np.uint32(32 - r))).astype(np.uint32)
            x1 = (x0 ^ x1).astype(np.uint32)
        a, b, inc = _SCHED[g]
        x0 = (x0 + ks[a]).astype(np.uint32)
        x1 = (x1 + ks[b] + np.uint32(inc)).astype(np.uint32)
    return (x0 ^ x1).astype(np.uint32)


def _np_uniform(seed_u32: int, n: int) -> np.ndarray:
    bits = _np_threefry_bits(seed_u32, np.arange(n, dtype=np.uint32))
    fb = ((bits >> np.uint32(9)) | np.uint32(0x3F800000)).astype(np.uint32)
    return fb.view(np.float32) - np.float32(1.0)


_COIN = float(_np_uniform(0, 1)[0])


def _tf_bits(cnt_u32, ks1, ks2):
    zero = jnp.zeros((LANES,), jnp.uint32)
    ks = (zero, ks1, ks2)
    x0 = zero
    x1 = cnt_u32 + ks1
    for g in range(5):
        for r in _ROTS[g % 2]:
            x0 = x0 + x1
            x1 = (x1 << jnp.uint32(r)) | (x1 >> jnp.uint32(32 - r))
            x1 = x0 ^ x1
        a, b, inc = _SCHED[g]
        x0 = x0 + ks[a]
        x1 = x1 + ks[b] + jnp.uint32(inc)
    return x0 ^ x1


def _bits_to_unif(bits):
    fb = (bits >> jnp.uint32(9)) | jnp.uint32(0x3F800000)
    return lax.bitcast_convert_type(fb, jnp.float32) - jnp.float32(1.0)


def _splat(x):
    return lax.broadcast(x, (LANES,))


def _any(mask):
    return jnp.max(jnp.where(mask, jnp.int32(1), jnp.int32(0))) == jnp.int32(1)


def _u32div3(x):
    q = (x >> jnp.uint32(2)) + (x >> jnp.uint32(4))
    q = q + (q >> jnp.uint32(4))
    q = q + (q >> jnp.uint32(8))
    q = q + (q >> jnp.uint32(16))
    r = x - (q + q + q)
    return q + ((r * jnp.uint32(11)) >> jnp.uint32(5))


def _sc_body(l_hbm, ids_hbm, out_hbm,
             lbuf, ebuf, idbuf, stg, cbuf, obuf, evbuf, avbuf, aibuf,
             dmasem, shared):
    t = lax.axis_index("s")
    base_t = t * MAIN
    iota_i = lax.iota(jnp.int32, LANES)
    iota_u = lax.bitcast_convert_type(iota_i, jnp.uint32)
    ninf = jnp.float32(-3.4e38)
    pinf = jnp.float32(np.inf)
    one = jnp.full((LANES,), jnp.uint32(1))
    zero_u = jnp.zeros((LANES,), jnp.uint32)
    maxint = jnp.full((LANES,), jnp.int32(0x7FFFFFFF))
    zr = jnp.int32(0)

    cp0 = pltpu.async_copy(l_hbm.at[zr, pl.ds(base_t, MAIN)], lbuf, dmasem)
    cp1 = pltpu.async_copy(l_hbm.at[zr, pl.ds(REM_BASE, REM_TOTAL)], ebuf,
                           dmasem)
    pltpu.sync_copy(ids_hbm, idbuf)

    w = lax.bitcast_convert_type(idbuf[...], jnp.uint32)
    hi = zero_u
    lo = jnp.full((LANES,), jnp.uint32(1))
    for lane in range(6, 16):
        b = _splat(jnp.max(jnp.where(iota_i == lane, w, zero_u)))
        l0 = lo & jnp.uint32(0xFFFF)
        l1 = lo >> jnp.uint32(16)
        p0 = l0 * b
        p1 = l1 * b
        new_lo = p0 + (p1 << jnp.uint32(16))
        carry = (p1 >> jnp.uint32(16)) + jnp.where(new_lo < p0, one, zero_u)
        hi = hi * b + carry
        lo = new_lo
    if _COIN < 0.5:
        seed = lo
    else:
        neg = (hi >> jnp.uint32(31)) == jnp.uint32(1)
        lo_m = jnp.where(neg, ~lo + jnp.uint32(1), lo)
        hi_m = jnp.where(neg,
                         ~hi + jnp.where(lo_m == jnp.uint32(0), one, zero_u),
                         hi)
        lo2 = jnp.where(neg, lo_m + jnp.uint32(2), lo_m)
        hi2 = jnp.where(neg & (lo2 < jnp.uint32(2)), hi_m + jnp.uint32(1), hi_m)
        q_hi = _u32div3(hi2)
        r = hi2 - (q_hi + q_hi + q_hi)
        tt = r + lo2
        wrapped = tt < lo2
        add_q = jnp.where(wrapped,
                          jnp.uint32(0x55555555) + _u32div3(tt + jnp.uint32(1)),
                          _u32div3(tt))
        q_lo = r * jnp.uint32(0x55555555) + add_q
        seed = jnp.where(neg, ~q_lo + jnp.uint32(1), q_lo)
    ks1 = seed
    ks2 = ks1 ^ jnp.uint32(0x1BD11BDA)

    cp0.wait()
    cp1.wait()

    rem_ids = [t * REM_PER_TILE + j for j in range(REM_PER_TILE)]
    rem_ok = [_splat(rv < REM_VECS) for rv in rem_ids]
    rem_ofs = [jnp.minimum(rv, REM_VECS - 1) * LANES for rv in rem_ids]

    def max_step(v, carry):
        mvs, mis, idx0 = carry
        off = v * (UNR * LANES)
        nm, ni = [], []
        for k in range(UNR):
            lv = lbuf[pl.ds(off + k * LANES, LANES)]
            cond = lv > mvs[k]
            nm.append(jnp.where(cond, lv, mvs[k]))
            ni.append(jnp.where(cond, idx0 + jnp.int32(k * LANES), mis[k]))
        return tuple(nm), tuple(ni), idx0 + jnp.int32(UNR * LANES)

    mvs0 = tuple(jnp.full((LANES,), ninf) for _ in range(UNR))
    mis0 = tuple(jnp.zeros((LANES,), jnp.int32) for _ in range(UNR))
    mvs, mis, _ = lax.fori_loop(jnp.int32(0), jnp.int32(MAIN_VECS // UNR),
                                max_step, (mvs0, mis0, _splat(base_t) + iota_i))
    mvec, mividx = mvs[0], mis[0]
    for k in range(1, UNR):
        cond = mvs[k] > mvec
        mvec = jnp.where(cond, mvs[k], mvec)
        mividx = jnp.where(cond, mis[k], mividx)
    evs, eidxs = [], []
    for j in range(REM_PER_TILE):
        ev = ebuf[pl.ds(rem_ofs[j], LANES)]
        eidx = _splat(REM_BASE + rem_ofs[j]) + iota_i
        evs.append(ev)
        eidxs.append(eidx)
        evm = jnp.where(rem_ok[j], ev, ninf)
        cond = evm > mvec
        mvec = jnp.where(cond, evm, mvec)
        mividx = jnp.where(cond, eidx, mividx)
    m_spl = _splat(jnp.max(mvec))

    def sum_step(v, svs):
        off = v * (UNR * LANES)
        return tuple(
            svs[k] + jnp.exp(lbuf[pl.ds(off + k * LANES, LANES)] - m_spl)
            for k in range(UNR))
    svs = lax.fori_loop(jnp.int32(0), jnp.int32(MAIN_VECS // UNR), sum_step,
                        tuple(jnp.zeros((LANES,), jnp.float32) for _ in range(UNR)))
    svec = (svs[0] + svs[1]) + (svs[2] + svs[3])
    for j in range(REM_PER_TILE):
        svec = svec + jnp.where(rem_ok[j], jnp.exp(evs[j] - m_spl),
                                jnp.float32(0.0))
    s_spl = _splat(jnp.sum(svec))

    stg[pl.ds(0, LANES)] = m_spl
    stg[pl.ds(LANES, LANES)] = s_spl
    stg[pl.ds(2 * LANES, LANES)] = mvec
    stg[pl.ds(3 * LANES, LANES)] = lax.bitcast_convert_type(mividx, jnp.float32)
    pltpu.sync_copy(stg, shared.at[pl.ds(t * SROW, SROW)])
    plsc.subcore_barrier()

    pltpu.sync_copy(shared, cbuf)
    gm = jnp.full((LANES,), ninf)
    for r in range(NTILES):
        gm = jnp.maximum(gm, cbuf[pl.ds(r * SROW, LANES)])
    S = jnp.zeros((LANES,), jnp.float32)
    for r in range(NTILES):
        S = S + (cbuf[pl.ds(r * SROW + LANES, LANES)]
                 * jnp.exp(cbuf[pl.ds(r * SROW, LANES)] - gm))
    sbits = lax.bitcast_convert_type(S, jnp.uint32)
    e_i = lax.bitcast_convert_type(sbits >> jnp.uint32(23), jnp.int32) - 127
    mant = lax.bitcast_convert_type(
        (sbits & jnp.uint32(0x7FFFFF)) | jnp.uint32(0x3F800000), jnp.float32)
    y = e_i.astype(jnp.float32) * jnp.float32(0.6931472) \
        + (mant - jnp.float32(1.0)) * jnp.float32(0.7)
    for _ in range(4):
        y = y + S * jnp.exp(-y) - jnp.float32(1.0)
    C = gm + y

    u = _bits_to_unif(_tf_bits(lax.bitcast_convert_type(mividx, jnp.uint32),
                               ks1, ks2))
    bval = (C - mvec) / u
    B = _splat(jnp.min(bval))

    neg = jnp.full((LANES,), jnp.float32(-100000.0))
    def fstep(v, _):
        off = v * (UNR * LANES)
        for k in range(UNR):
            obuf[pl.ds(off + k * LANES, LANES)] = neg
        return zr
    lax.fori_loop(jnp.int32(0), jnp.int32(MAIN_VECS // UNR), fstep, zr)
    pltpu.sync_copy(obuf, out_hbm.at[zr, pl.ds(base_t, MAIN)])

    @pl.when(t == 0)
    def _():
        pltpu.sync_copy(obuf.at[pl.ds(0, REM_TOTAL)],
                        out_hbm.at[zr, pl.ds(REM_BASE, REM_TOTAL)])

    avbuf[...] = jnp.full((LANES,), pinf)
    aibuf[...] = maxint
    base_u = lax.bitcast_convert_type(base_t, jnp.uint32)

    def hit_update(d, hit, cnt_u, idx):
        u = _bits_to_unif(_tf_bits(cnt_u, ks1, ks2))
        val = jnp.where(hit, d / u, pinf)
        vidx = jnp.where(hit, idx, maxint)
        av = avbuf[...]
        ai = aibuf[...]
        better = (val < av) | ((val == av) & (vidx < ai))
        avbuf[...] = jnp.where(better, val, av)
        aibuf[...] = jnp.where(better, vidx, ai)

    def bstep(v, carry):
        cnt0, idx0 = carry
        off = v * (UNR * LANES)
        lvs = [lbuf[pl.ds(off + k * LANES, LANES)] for k in range(UNR)]
        ds = [C - lv for lv in lvs]
        hits = [d <= B for d in ds]
        anyhit = (hits[0] | hits[1]) | (hits[2] | hits[3])

        @pl.when(_any(anyhit))
        def _():
            for k in range(UNR):
                @pl.when(_any(hits[k]))
                def _(k=k):
                    hit_update(ds[k], hits[k],
                               cnt0 + jnp.uint32(k * LANES),
                               idx0 + jnp.int32(k * LANES))
        return cnt0 + jnp.uint32(UNR * LANES), idx0 + jnp.int32(UNR * LANES)

    lax.fori_loop(jnp.int32(0), jnp.int32(MAIN_VECS // UNR), bstep,
                  (_splat(base_u) + iota_u, _splat(base_t) + iota_i))

    for j in range(REM_PER_TILE):
        ed = jnp.where(rem_ok[j], C - evs[j], pinf)
        ehit = ed <= B

        @pl.when(_any(ehit))
        def _(j=j, ed=ed, ehit=ehit):
            hit_update(ed, ehit,
                       lax.bitcast_convert_type(eidxs[j], jnp.uint32), eidxs[j])

    stg[pl.ds(0, LANES)] = avbuf[...]
    stg[pl.ds(LANES, LANES)] = lax.bitcast_convert_type(aibuf[...], jnp.float32)
    pltpu.sync_copy(stg.at[pl.ds(0, 2 * LANES)],
                    shared.at[pl.ds(t * SROW, 2 * LANES)])
    plsc.subcore_barrier()

    pltpu.sync_copy(shared, cbuf)
    bv = jnp.full((LANES,), pinf)
    bi = maxint
    for r in range(NTILES):
        vr = cbuf[pl.ds(r * SROW, LANES)]
        ir = lax.bitcast_convert_type(cbuf[pl.ds(r * SROW + LANES, LANES)],
                                      jnp.int32)
        cond = (vr < bv) | ((vr == bv) & (ir < bi))
        bv = jnp.where(cond, vr, bv)
        bi = jnp.where(cond, ir, bi)
    mv = _splat(jnp.min(bv))
    cand = jnp.where(bv == mv, bi, maxint)
    token = _splat(jnp.minimum(jnp.min(cand), jnp.int32(VOCAB - 1)))

    pos = jnp.full((LANES,), jnp.float32(100000.0))
    tbase = token - (token % jnp.int32(128))

    @pl.when(t == 0)
    def _():
        for k in range(8):
            g = tbase + jnp.int32(k * LANES) + iota_i
            evbuf[pl.ds(k * LANES, LANES)] = jnp.where(g == token, pos, neg)
        tb_s = jnp.min(tbase)

        @pl.when(tb_s < jnp.int32(REM_LAST))
        def _():
            tb0 = pl.multiple_of(tb_s, 128)
            pltpu.sync_copy(evbuf, out_hbm.at[zr, pl.ds(tb0, 128)])

        @pl.when(tb_s == jnp.int32(REM_LAST))
        def _():
            pltpu.sync_copy(evbuf.at[pl.ds(0, VOCAB - REM_LAST)],
                            out_hbm.at[zr, pl.ds(REM_LAST, VOCAB - REM_LAST)])


def _sc_call(logits, ids_tail):
    mesh = plsc.VectorSubcoreMesh(
        core_axis_name="c", subcore_axis_name="s", num_cores=1)
    f = pl.kernel(
        _sc_body,
        out_type=jax.ShapeDtypeStruct((1, VOCAB), jnp.float32),
        mesh=mesh,
        scratch_types=[
            pltpu.VMEM((MAIN,), jnp.float32),
            pltpu.VMEM((REM_TOTAL,), jnp.float32),
            pltpu.VMEM((LANES,), jnp.int32),
            pltpu.VMEM((SROW,), jnp.float32),
            pltpu.VMEM((NTILES * SROW,), jnp.float32),
            pltpu.VMEM((MAIN,), jnp.float32),
            pltpu.VMEM((128,), jnp.float32),
            pltpu.VMEM((LANES,), jnp.float32),
            pltpu.VMEM((LANES,), jnp.int32),
            pltpu.SemaphoreType.DMA,
            pltpu.VMEM_SHARED((NTILES * SROW,), jnp.float32),
        ],
        compiler_params=pltpu.CompilerParams(needs_layout_passes=False,
                                             use_tc_tiling_on_sc=True),
    )
    return f(logits, ids_tail)


@jax.jit
def kernel(input_ids, logits):
    ids_tail = lax.convert_element_type(
        input_ids.reshape(-1)[-LANES:], jnp.int32)
    return _sc_call(logits, ids_tail)


# speedup vs baseline: 1.4114x; 1.4114x over previous
import numpy as np
import jax
import jax.numpy as jnp
from jax import lax
from jax.experimental import pallas as pl
from jax.experimental.pallas import tpu as pltpu
from jax.experimental.pallas import tpu_sc as plsc

VOCAB = 50272
LANES = 16
NTILES = 16
UNR = 4
MAIN_VECS = 192
MAIN = MAIN_VECS * LANES
MAIN_TOTAL = NTILES * MAIN
REM_BASE = MAIN_TOTAL
REM_TOTAL = VOCAB - MAIN_TOTAL
REM_VECS = REM_TOTAL // LANES
REM_PER_TILE = 5
REM_LAST = (VOCAB // 128) * 128
SROW = 4 * LANES

_ROTS = ((13, 15, 26, 6), (17, 29, 16, 24))
_SCHED = ((1, 2, 1), (2, 0, 2), (0, 1, 3), (1, 2, 4), (2, 0, 5))


def _np_threefry_bits(seed_u32: int, counts_lo: np.ndarray) -> np.ndarray:
    x0 = np.zeros_like(counts_lo, dtype=np.uint32)
    x1 = counts_lo.astype(np.uint32)
    ks = [np.uint32(0), np.uint32(seed_u32),
          np.uint32(np.uint32(seed_u32) ^ np.uint32(0x1BD11BDA))]
    x0 = (x0 + ks[0]).astype(np.uint32)
    x1 = (x1 + ks[1]).astype(np.uint32)
    for g in range(5):
        for r in _ROTS[g % 2]:
            x0 = (x0 + x1).astype(np.uint32)
            x1 = ((x1 << np.uint32(r)) | (x1 >> np.uint32(32 - r))).astype(np.uint32)
            x1 = (x0 ^ x1).astype(np.uint32)
        a, b, inc = _SCHED[g]
        x0 = (x0 + ks[a]).astype(np.uint32)
        x1 = (x1 + ks[b] + np.uint32(inc)).astype(np.uint32)
    return (x0 ^ x1).astype(np.uint32)


def _np_uniform(seed_u32: int, n: int) -> np.ndarray:
    bits = _np_threefry_bits(seed_u32, np.arange(n, dtype=np.uint32))
    fb = ((bits >> np.uint32(9)) | np.uint32(0x3F800000)).astype(np.uint32)
    return fb.view(np.float32) - np.float32(1.0)


_COIN = float(_np_uniform(0, 1)[0])


def _tf_bits(cnt_u32, ks1, ks2):
    zero = jnp.zeros((LANES,), jnp.uint32)
    ks = (zero, ks1, ks2)
    x0 = zero
    x1 = cnt_u32 + ks1
    for g in range(5):
        for r in _ROTS[g % 2]:
            x0 = x0 + x1
            x1 = (x1 << jnp.uint32(r)) | (x1 >> jnp.uint32(32 - r))
            x1 = x0 ^ x1
        a, b, inc = _SCHED[g]
        x0 = x0 + ks[a]
        x1 = x1 + ks[b] + jnp.uint32(inc)
    return x0 ^ x1


def _bits_to_unif(bits):
    fb = (bits >> jnp.uint32(9)) | jnp.uint32(0x3F800000)
    return lax.bitcast_convert_type(fb, jnp.float32) - jnp.float32(1.0)


def _splat(x):
    return lax.broadcast(x, (LANES,))


def _any(mask):
    return jnp.max(jnp.where(mask, jnp.int32(1), jnp.int32(0))) == jnp.int32(1)


def _u32div3(x):
    q = (x >> jnp.uint32(2)) + (x >> jnp.uint32(4))
    q = q + (q >> jnp.uint32(4))
    q = q + (q >> jnp.uint32(8))
    q = q + (q >> jnp.uint32(16))
    r = x - (q + q + q)
    return q + ((r * jnp.uint32(11)) >> jnp.uint32(5))


def _sc_body(l_hbm, ids_hbm, out_hbm,
             lbuf, ebuf, idbuf, stg, cbuf, obuf, evbuf, avbuf, aibuf,
             dmasem, shared):
    t = lax.axis_index("s")
    base_t = t * MAIN
    iota_i = lax.iota(jnp.int32, LANES)
    iota_u = lax.bitcast_convert_type(iota_i, jnp.uint32)
    ninf = jnp.float32(-3.4e38)
    pinf = jnp.float32(np.inf)
    one = jnp.full((LANES,), jnp.uint32(1))
    zero_u = jnp.zeros((LANES,), jnp.uint32)
    maxint = jnp.full((LANES,), jnp.int32(0x7FFFFFFF))
    zr = jnp.int32(0)

    cp0 = pltpu.async_copy(l_hbm.at[zr, pl.ds(base_t, MAIN)], lbuf, dmasem)
    cp1 = pltpu.async_copy(l_hbm.at[zr, pl.ds(REM_BASE, REM_TOTAL)], ebuf,
                           dmasem)
    pltpu.sync_copy(ids_hbm, idbuf)

    w = lax.bitcast_convert_type(idbuf[...], jnp.uint32)
    hi = zero_u
    lo = jnp.full((LANES,), jnp.uint32(1))
    for lane in range(6, 16):
        b = _splat(jnp.max(jnp.where(iota_i == lane, w, zero_u)))
        l0 = lo & jnp.uint32(0xFFFF)
        l1 = lo >> jnp.uint32(16)
        p0 = l0 * b
        p1 = l1 * b
        new_lo = p0 + (p1 << jnp.uint32(16))
        carry = (p1 >> jnp.uint32(16)) + jnp.where(new_lo < p0, one, zero_u)
        hi = hi * b + carry
        lo = new_lo
    if _COIN < 0.5:
        seed = lo
    else:
        neg = (hi >> jnp.uint32(31)) == jnp.uint32(1)
        lo_m = jnp.where(neg, ~lo + jnp.uint32(1), lo)
        hi_m = jnp.where(neg,
                         ~hi + jnp.where(lo_m == jnp.uint32(0), one, zero_u),
                         hi)
        lo2 = jnp.where(neg, lo_m + jnp.uint32(2), lo_m)
        hi2 = jnp.where(neg & (lo2 < jnp.uint32(2)), hi_m + jnp.uint32(1), hi_m)
        q_hi = _u32div3(hi2)
        r = hi2 - (q_hi + q_hi + q_hi)
        tt = r + lo2
        wrapped = tt < lo2
        add_q = jnp.where(wrapped,
                          jnp.uint32(0x55555555) + _u32div3(tt + jnp.uint32(1)),
                          _u32div3(tt))
        q_lo = r * jnp.uint32(0x55555555) + add_q
        seed = jnp.where(neg, ~q_lo + jnp.uint32(1), q_lo)
    ks1 = seed
    ks2 = ks1 ^ jnp.uint32(0x1BD11BDA)

    cp0.wait()
    cp1.wait()

    rem_ids = [t * REM_PER_TILE + j for j in range(REM_PER_TILE)]
    rem_ok = [_splat(rv < REM_VECS) for rv in rem_ids]
    rem_ofs = [jnp.minimum(rv, REM_VECS - 1) * LANES for rv in rem_ids]

    def max_step(v, carry):
        mvs, mis, idx0 = carry
        off = v * (UNR * LANES)
        nm, ni = [], []
        for k in range(UNR):
            lv = lbuf[pl.ds(off + k * LANES, LANES)]
            cond = lv > mvs[k]
            nm.append(jnp.where(cond, lv, mvs[k]))
            ni.append(jnp.where(cond, idx0 + jnp.int32(k * LANES), mis[k]))
        return tuple(nm), tuple(ni), idx0 + jnp.int32(UNR * LANES)

    mvs0 = tuple(jnp.full((LANES,), ninf) for _ in range(UNR))
    mis0 = tuple(jnp.zeros((LANES,), jnp.int32) for _ in range(UNR))
    mvs, mis, _ = lax.fori_loop(jnp.int32(0), jnp.int32(MAIN_VECS // UNR),
                                max_step, (mvs0, mis0, _splat(base_t) + iota_i))
    mvec, mividx = mvs[0], mis[0]
    for k in range(1, UNR):
        cond = mvs[k] > mvec
        mvec = jnp.where(cond, mvs[k], mvec)
        mividx = jnp.where(cond, mis[k], mividx)
    evs, eidxs = [], []
    for j in range(REM_PER_TILE):
        ev = ebuf[pl.ds(rem_ofs[j], LANES)]
        eidx = _splat(REM_BASE + rem_ofs[j]) + iota_i
        evs.append(ev)
        eidxs.append(eidx)
        evm = jnp.where(rem_ok[j], ev, ninf)
        cond = evm > mvec
        mvec = jnp.where(cond, evm, mvec)
        mividx = jnp.where(cond, eidx, mividx)
    m_spl = _splat(jnp.max(mvec))

    def sum_step(v, svs):
        off = v * (UNR * LANES)
        return tuple(
            svs[k] + jnp.exp(lbuf[pl.ds(off + k * LANES, LANES)] - m_spl)
            for k in range(UNR))
    svs = lax.fori_loop(jnp.int32(0), jnp.int32(MAIN_VECS // UNR), sum_step,
                        tuple(jnp.zeros((LANES,), jnp.float32) for _ in range(UNR)))
    svec = (svs[0] + svs[1]) + (svs[2] + svs[3])
    for j in range(REM_PER_TILE):
        svec = svec + jnp.where(rem_ok[j], jnp.exp(evs[j] - m_spl),
                                jnp.float32(0.0))
    s_spl = _splat(jnp.sum(svec))

    stg[pl.ds(0, LANES)] = m_spl
    stg[pl.ds(LANES, LANES)] = s_spl
    stg[pl.ds(2 * LANES, LANES)] = mvec
    stg[pl.ds(3 * LANES, LANES)] = lax.bitcast_convert_type(mividx, jnp.float32)
    pltpu.sync_copy(stg, shared.at[pl.ds(t * SROW, SROW)])
    plsc.subcore_barrier()

    pltpu.sync_copy(shared, cbuf)
    gm = jnp.full((LANES,), ninf)
    for r in range(NTILES):
        gm = jnp.maximum(gm, cbuf[pl.ds(r * SROW, LANES)])
    S = jnp.zeros((LANES,), jnp.float32)
    for r in range(NTILES):
        S = S + (cbuf[pl.ds(r * SROW + LANES, LANES)]
                 * jnp.exp(cbuf[pl.ds(r * SROW, LANES)] - gm))
    sbits = lax.bitcast_convert_type(S, jnp.uint32)
    e_i = lax.bitcast_convert_type(sbits >> jnp.uint32(23), jnp.int32) - 127
    mant = lax.bitcast_convert_type(
        (sbits & jnp.uint32(0x7FFFFF)) | jnp.uint32(0x3F800000), jnp.float32)
    y = e_i.astype(jnp.float32) * jnp.float32(0.6931472) \
        + (mant - jnp.float32(1.0)) * jnp.float32(0.7)
    for _ in range(4):
        y = y + S * jnp.exp(-y) - jnp.float32(1.0)
    C = gm + y

    u = _bits_to_unif(_tf_bits(lax.bitcast_convert_type(mividx, jnp.uint32),
                               ks1, ks2))
    bval = (C - mvec) / u
    stg[pl.ds(0, LANES)] = _splat(jnp.min(bval))
    pltpu.sync_copy(stg.at[pl.ds(0, LANES)], shared.at[pl.ds(t * SROW, LANES)])

    neg = jnp.full((LANES,), jnp.float32(-100000.0))
    def fstep(v, _):
        off = v * (UNR * LANES)
        for k in range(UNR):
            obuf[pl.ds(off + k * LANES, LANES)] = neg
        return zr
    lax.fori_loop(jnp.int32(0), jnp.int32(MAIN_VECS // UNR), fstep, zr)
    pltpu.sync_copy(obuf, out_hbm.at[zr, pl.ds(base_t, MAIN)])

    @pl.when(t == 0)
    def _():
        pltpu.sync_copy(obuf.at[pl.ds(0, REM_TOTAL)],
                        out_hbm.at[zr, pl.ds(REM_BASE, REM_TOTAL)])

    plsc.subcore_barrier()

    pltpu.sync_copy(shared, cbuf)
    B = jnp.full((LANES,), pinf)
    for r in range(NTILES):
        B = jnp.minimum(B, cbuf[pl.ds(r * SROW, LANES)])

    avbuf[...] = jnp.full((LANES,), pinf)
    aibuf[...] = maxint
    base_u = lax.bitcast_convert_type(base_t, jnp.uint32)

    def hit_update(d, hit, cnt_u, idx):
        u = _bits_to_unif(_tf_bits(cnt_u, ks1, ks2))
        val = jnp.where(hit, d / u, pinf)
        vidx = jnp.where(hit, idx, maxint)
        av = avbuf[...]
        ai = aibuf[...]
        better = (val < av) | ((val == av) & (vidx < ai))
        avbuf[...] = jnp.where(better, val, av)
        aibuf[...] = jnp.where(better, vidx, ai)

    def bstep(v, carry):
        cnt0, idx0 = carry
        off = v * (UNR * LANES)
        lvs = [lbuf[pl.ds(off + k * LANES, LANES)] for k in range(UNR)]
        ds = [C - lv for lv in lvs]
        hits = [d <= B for d in ds]
        anyhit = (hits[0] | hits[1]) | (hits[2] | hits[3])

        @pl.when(_any(anyhit))
        def _():
            for k in range(UNR):
                @pl.when(_any(hits[k]))
                def _(k=k):
                    hit_update(ds[k], hits[k],
                               cnt0 + jnp.uint32(k * LANES),
                               idx0 + jnp.int32(k * LANES))
        return cnt0 + jnp.uint32(UNR * LANES), idx0 + jnp.int32(UNR * LANES)

    lax.fori_loop(jnp.int32(0), jnp.int32(MAIN_VECS // UNR), bstep,
                  (_splat(base_u) + iota_u, _splat(base_t) + iota_i))

    for j in range(REM_PER_TILE):
        ed = jnp.where(rem_ok[j], C - evs[j], pinf)
        ehit = ed <= B

        @pl.when(_any(ehit))
        def _(j=j, ed=ed, ehit=ehit):
            hit_update(ed, ehit,
                       lax.bitcast_convert_type(eidxs[j], jnp.uint32), eidxs[j])

    stg[pl.ds(0, LANES)] = avbuf[...]
    stg[pl.ds(LANES, LANES)] = lax.bitcast_convert_type(aibuf[...], jnp.float32)
    pltpu.sync_copy(stg.at[pl.ds(0, 2 * LANES)],
                    shared.at[pl.ds(t * SROW, 2 * LANES)])
    plsc.subcore_barrier()

    pltpu.sync_copy(shared, cbuf)
    bv = jnp.full((LANES,), pinf)
    bi = maxint
    for r in range(NTILES):
        vr = cbuf[pl.ds(r * SROW, LANES)]
        ir = lax.bitcast_convert_type(cbuf[pl.ds(r * SROW + LANES, LANES)],
                                      jnp.int32)
        cond = (vr < bv) | ((vr == bv) & (ir < bi))
        bv = jnp.where(cond, vr, bv)
        bi = jnp.where(cond, ir, bi)
    mv = _splat(jnp.min(bv))
    cand = jnp.where(bv == mv, bi, maxint)
    token = _splat(jnp.minimum(jnp.min(cand), jnp.int32(VOCAB - 1)))

    pos = jnp.full((LANES,), jnp.float32(100000.0))
    tbase = token - (token % jnp.int32(128))

    @pl.when(t == 0)
    def _():
        for k in range(8):
            g = tbase + jnp.int32(k * LANES) + iota_i
            evbuf[pl.ds(k * LANES, LANES)] = jnp.where(g == token, pos, neg)
        tb_s = jnp.min(tbase)

        @pl.when(tb_s < jnp.int32(REM_LAST))
        def _():
            tb0 = pl.multiple_of(tb_s, 128)
            pltpu.sync_copy(evbuf, out_hbm.at[zr, pl.ds(tb0, 128)])

        @pl.when(tb_s == jnp.int32(REM_LAST))
        def _():
            pltpu.sync_copy(evbuf.at[pl.ds(0, VOCAB - REM_LAST)],
                            out_hbm.at[zr, pl.ds(REM_LAST, VOCAB - REM_LAST)])


def _sc_call(logits, ids_tail):
    mesh = plsc.VectorSubcoreMesh(
        core_axis_name="c", subcore_axis_name="s", num_cores=1)
    f = pl.kernel(
        _sc_body,
        out_type=jax.ShapeDtypeStruct((1, VOCAB), jnp.float32),
        mesh=mesh,
        scratch_types=[
            pltpu.VMEM((MAIN,), jnp.float32),
            pltpu.VMEM((REM_TOTAL,), jnp.float32),
            pltpu.VMEM((LANES,), jnp.int32),
            pltpu.VMEM((SROW,), jnp.float32),
            pltpu.VMEM((NTILES * SROW,), jnp.float32),
            pltpu.VMEM((MAIN,), jnp.float32),
            pltpu.VMEM((128,), jnp.float32),
            pltpu.VMEM((LANES,), jnp.float32),
            pltpu.VMEM((LANES,), jnp.int32),
            pltpu.SemaphoreType.DMA,
            pltpu.VMEM_SHARED((NTILES * SROW,), jnp.float32),
        ],
        compiler_params=pltpu.CompilerParams(needs_layout_passes=False,
                                             use_tc_tiling_on_sc=True),
    )
    return f(logits, ids_tail)


@jax.jit
def kernel(input_ids, logits):
    ids_tail = lax.convert_element_type(
        input_ids.reshape(-1)[-LANES:], jnp.int32)
    return _sc_call(logits, ids_tail)
